# baseline (speedup 1.0000x reference)
import functools

import jax
import jax.numpy as jnp
from jax import lax
from jax.experimental import pallas as pl
from jax.experimental.pallas import tpu as pltpu

N = 50000
NE = 800000
D_IN = 64
D_E = 16
H = 128
L = 4
D_OUT = 32

NBLK = 1000
EBLK = 8000


def _ln(x, g, b, eps=1e-5):
    m = x.mean(-1, keepdims=True)
    v = ((x - m) ** 2).mean(-1, keepdims=True)
    return (x - m) * lax.rsqrt(v + eps) * g + b


def _enc_body(x_ref, w1_ref, b1_ref, g_ref, be_ref, w2_ref, b2_ref, o_ref):
    x = x_ref[...]
    h = jnp.maximum(x @ w1_ref[...] + b1_ref[...], 0.0)
    h = _ln(h, g_ref[...], be_ref[...])
    o_ref[...] = jnp.maximum(h @ w2_ref[...] + b2_ref[...], 0.0)


def _encoder(x, w1, b1, g, be, w2, b2):
    grid = (N // NBLK,)
    full = lambda r, c: pl.BlockSpec((r, c), lambda i: (0, 0))
    return pl.pallas_call(
        _enc_body,
        grid=grid,
        in_specs=[
            pl.BlockSpec((NBLK, D_IN), lambda i: (i, 0)),
            full(D_IN, H), full(1, H), full(1, H), full(1, H),
            full(H, H), full(1, H),
        ],
        out_specs=pl.BlockSpec((NBLK, H), lambda i: (i, 0)),
        out_shape=jax.ShapeDtypeStruct((N, H), jnp.float32),
    )(x, w1, b1.reshape(1, H), g.reshape(1, H), be.reshape(1, H),
      w2, b2.reshape(1, H))


def _ab_body(h_ref, w_ref, b_ref, a_ref, bo_ref):
    h = h_ref[...]
    ab = h @ w_ref[...] + b_ref[...]
    a_ref[...] = ab[:, :H]
    bo_ref[...] = ab[:, H:]


def _ab(h, w_ab, bias_ab):
    grid = (N // NBLK,)
    return pl.pallas_call(
        _ab_body,
        grid=grid,
        in_specs=[
            pl.BlockSpec((NBLK, H), lambda i: (i, 0)),
            pl.BlockSpec((H, 2 * H), lambda i: (0, 0)),
            pl.BlockSpec((1, 2 * H), lambda i: (0, 0)),
        ],
        out_specs=[
            pl.BlockSpec((NBLK, H), lambda i: (i, 0)),
            pl.BlockSpec((NBLK, H), lambda i: (i, 0)),
        ],
        out_shape=[
            jax.ShapeDtypeStruct((N, H), jnp.float32),
            jax.ShapeDtypeStruct((N, H), jnp.float32),
        ],
    )(h, w_ab, bias_ab)


def _e_body(e_ref, w_ref, o_ref):
    o_ref[...] = e_ref[...] @ w_ref[...]


def _eproj(ef, w_e):
    grid = (NE // EBLK,)
    return pl.pallas_call(
        _e_body,
        grid=grid,
        in_specs=[
            pl.BlockSpec((EBLK, D_E), lambda i: (i, 0)),
            pl.BlockSpec((D_E, H), lambda i: (0, 0)),
        ],
        out_specs=pl.BlockSpec((EBLK, H), lambda i: (i, 0)),
        out_shape=jax.ShapeDtypeStruct((NE, H), jnp.float32),
    )(ef, w_e)


def _upd_body(h_ref, p_ref, cnt_ref, w2_ref, b2_ref, wuh_ref, wua_ref,
              ub_ref, g_ref, b_ref, o_ref):
    cnt = cnt_ref[...]
    agg = (p_ref[...] @ w2_ref[...] + cnt * b2_ref[...]) / (cnt + 1e-8)
    h = h_ref[...]
    hn = jnp.maximum(h @ wuh_ref[...] + agg @ wua_ref[...] + ub_ref[...], 0.0)
    o_ref[...] = _ln(h + hn, g_ref[...], b_ref[...])


def _update(h, p, cnt, w2, b2, wuh, wua, ub, g, b):
    grid = (N // NBLK,)
    full = lambda r, c: pl.BlockSpec((r, c), lambda i: (0, 0))
    return pl.pallas_call(
        _upd_body,
        grid=grid,
        in_specs=[
            pl.BlockSpec((NBLK, H), lambda i: (i, 0)),
            pl.BlockSpec((NBLK, H), lambda i: (i, 0)),
            pl.BlockSpec((NBLK, 1), lambda i: (i, 0)),
            full(H, H), full(1, H), full(H, H), full(H, H),
            full(1, H), full(1, H), full(1, H),
        ],
        out_specs=pl.BlockSpec((NBLK, H), lambda i: (i, 0)),
        out_shape=jax.ShapeDtypeStruct((N, H), jnp.float32),
    )(h, p, cnt, w2, b2.reshape(1, H), wuh, wua,
      ub.reshape(1, H), g.reshape(1, H), b.reshape(1, H))


def _pool_body(h_ref, aw1_ref, ab1_ref, aw2_ref, ab2_ref,
               ow1_ref, ob1_ref, ow2_ref, ob2_ref, o_ref,
               sexp_ref, gacc_ref):
    i = pl.program_id(0)

    @pl.when(i == 0)
    def _():
        sexp_ref[0, 0] = 0.0
        gacc_ref[...] = jnp.zeros_like(gacc_ref)

    h = h_ref[...]
    t = jnp.tanh(h @ aw1_ref[...] + ab1_ref[...])
    att = t @ aw2_ref[...] + ab2_ref[...]
    w = jnp.exp(att)
    sexp_ref[0, 0] += jnp.sum(w)
    gacc_ref[...] += (w * h).sum(axis=0, keepdims=True)

    @pl.when(i == pl.num_programs(0) - 1)
    def _():
        g = gacc_ref[...] / sexp_ref[0, 0]
        o_ref[...] = jnp.maximum(g @ ow1_ref[...] + ob1_ref[...], 0.0) \
            @ ow2_ref[...] + ob2_ref[...]


def _pool_head(h, aw1, ab1, aw2, ab2, ow1, ob1, ow2, ob2):
    grid = (N // NBLK,)
    full = lambda r, c: pl.BlockSpec((r, c), lambda i: (0, 0))
    return pl.pallas_call(
        _pool_body,
        grid=grid,
        in_specs=[
            pl.BlockSpec((NBLK, H), lambda i: (i, 0)),
            full(H, H // 2), full(1, H // 2), full(H // 2, 1), full(1, 1),
            full(H, H), full(1, H), full(H, D_OUT), full(1, D_OUT),
        ],
        out_specs=pl.BlockSpec((1, D_OUT), lambda i: (0, 0)),
        out_shape=jax.ShapeDtypeStruct((1, D_OUT), jnp.float32),
        scratch_shapes=[
            pltpu.SMEM((1, 1), jnp.float32),
            pltpu.VMEM((1, H), jnp.float32),
        ],
    )(h, aw1, ab1.reshape(1, H // 2), aw2, ab2.reshape(1, 1),
      ow1, ob1.reshape(1, H), ow2, ob2.reshape(1, D_OUT))


def kernel(node_features, edge_index, edge_features, constraint_types,
           enc_W1, enc_b1, enc_g, enc_be, enc_W2, enc_b2,
           msg_W1, msg_b1, msg_W2, msg_b2, upd_W, upd_b, ln_g, ln_b,
           att_W1, att_b1, att_W2, att_b2,
           out_W1, out_b1, out_W2, out_b2):
    src = edge_index[0]
    dst = edge_index[1]

    h = _encoder(node_features, enc_W1, enc_b1, enc_g, enc_be, enc_W2, enc_b2)

    cnt = jnp.zeros((N, 1), jnp.float32).at[dst].add(
        jnp.ones((NE, 1), jnp.float32))

    for i in range(L):
        w1 = msg_W1[i]
        w_ab = jnp.concatenate([w1[:H], w1[H:2 * H]], axis=1)
        bias_ab = jnp.concatenate(
            [msg_b1[i], jnp.zeros_like(msg_b1[i])]).reshape(1, 2 * H)
        a, b = _ab(h, w_ab, bias_ab)
        e = _eproj(edge_features, w1[2 * H:])
        r = jnp.maximum(a[src] + b[dst] + e, 0.0)
        p = jnp.zeros((N, H), jnp.float32).at[dst].add(r)
        h = _update(h, p, cnt, msg_W2[i], msg_b2[i],
                    upd_W[i][:H], upd_W[i][H:], upd_b[i], ln_g[i], ln_b[i])

    out = _pool_head(h, att_W1, att_b1, att_W2, att_b2,
                     out_W1, out_b1, out_W2, out_b2)
    return out.reshape(D_OUT)
